# baseline (speedup 1.0000x reference)
import functools

import jax
import jax.numpy as jnp
from jax.experimental import pallas as pl

B, N, F, H = 2, 4096, 128, 128
BM = 512


def _proj_kernel(x_ref, w_ref, o_ref):
    o_ref[...] = jnp.dot(
        x_ref[0], w_ref[...], preferred_element_type=jnp.float32
    )[None]


def _layer1_kernel(adj_ref, s_ref, b_ref, w2_ref, s2_ref, gh_ref):
    i = pl.program_id(1)
    t = jnp.dot(
        adj_ref[0].astype(jnp.bfloat16),
        s_ref[0].astype(jnp.bfloat16),
        preferred_element_type=jnp.float32,
    )
    h = jnp.maximum(t + b_ref[...], 0.0)
    gh_part = jnp.sum(h, axis=0, keepdims=True)[None]

    @pl.when(i == 0)
    def _():
        gh_ref[...] = gh_part

    @pl.when(i != 0)
    def _():
        gh_ref[...] += gh_part

    s2_ref[...] = jnp.dot(h, w2_ref[...], preferred_element_type=jnp.float32)[None]


def _layer2_kernel(adj_ref, s_ref, b_ref, h_ref, gh_ref):
    i = pl.program_id(1)
    t = jnp.dot(
        adj_ref[0].astype(jnp.bfloat16),
        s_ref[0].astype(jnp.bfloat16),
        preferred_element_type=jnp.float32,
    )
    h = jnp.maximum(t + b_ref[...], 0.0)
    gh_part = jnp.sum(h, axis=0, keepdims=True)[None]

    @pl.when(i == 0)
    def _():
        gh_ref[...] = gh_part

    @pl.when(i != 0)
    def _():
        gh_ref[...] += gh_part

    h_ref[...] = h[None]


@functools.partial(jax.jit, static_argnames=("interpret",))
def _encoder(x, adj, W1, b1, W2, b2, interpret=False):
    b1r = b1.reshape(1, H)
    b2r = b2.reshape(1, H)

    s1 = pl.pallas_call(
        _proj_kernel,
        grid=(B,),
        in_specs=[
            pl.BlockSpec((1, N, F), lambda b: (b, 0, 0)),
            pl.BlockSpec((F, H), lambda b: (0, 0)),
        ],
        out_specs=pl.BlockSpec((1, N, H), lambda b: (b, 0, 0)),
        out_shape=jax.ShapeDtypeStruct((B, N, H), jnp.float32),
        interpret=interpret,
    )(x, W1)

    num_i = N // BM
    s2, gh1 = pl.pallas_call(
        _layer1_kernel,
        grid=(B, num_i),
        in_specs=[
            pl.BlockSpec((1, BM, N), lambda b, i: (b, i, 0)),
            pl.BlockSpec((1, N, H), lambda b, i: (b, 0, 0)),
            pl.BlockSpec((1, H), lambda b, i: (0, 0)),
            pl.BlockSpec((H, H), lambda b, i: (0, 0)),
        ],
        out_specs=[
            pl.BlockSpec((1, BM, H), lambda b, i: (b, i, 0)),
            pl.BlockSpec((1, 1, H), lambda b, i: (b, 0, 0)),
        ],
        out_shape=[
            jax.ShapeDtypeStruct((B, N, H), jnp.float32),
            jax.ShapeDtypeStruct((B, 1, H), jnp.float32),
        ],
        interpret=interpret,
    )(adj, s1, b1r, W2)

    h2, gh2 = pl.pallas_call(
        _layer2_kernel,
        grid=(B, num_i),
        in_specs=[
            pl.BlockSpec((1, BM, N), lambda b, i: (b, i, 0)),
            pl.BlockSpec((1, N, H), lambda b, i: (b, 0, 0)),
            pl.BlockSpec((1, H), lambda b, i: (0, 0)),
        ],
        out_specs=[
            pl.BlockSpec((1, BM, H), lambda b, i: (b, i, 0)),
            pl.BlockSpec((1, 1, H), lambda b, i: (b, 0, 0)),
        ],
        out_shape=[
            jax.ShapeDtypeStruct((B, N, H), jnp.float32),
            jax.ShapeDtypeStruct((B, 1, H), jnp.float32),
        ],
        interpret=interpret,
    )(adj, s2, b2r)

    gh = jnp.concatenate([gh1[:, 0, :], gh2[:, 0, :]], axis=-1)
    return h2, gh


def kernel(x, adj, W1, b1, W2, b2):
    return _encoder(x, adj, W1, b1, W2, b2)


# speedup vs baseline: 1.3370x; 1.3370x over previous
import functools

import jax
import jax.numpy as jnp
from jax.experimental import pallas as pl
from jax.experimental.pallas import tpu as pltpu

B, N, F, H = 2, 4096, 128, 128
BM = 512
NUM_I = N // BM


def _fused_kernel(adj_ref, x_ref, w1_ref, b1_ref, w2_ref, b2_ref,
                  h2_ref, gh_ref, s1_scr, s2_scr, cache_scr):
    p = pl.program_id(1)
    i = pl.program_id(2)

    @pl.when((p == 0) & (i == 0))
    def _():
        s1 = jnp.dot(x_ref[0], w1_ref[...], preferred_element_type=jnp.float32)
        s1_scr[...] = s1.astype(jnp.bfloat16)

    @pl.when(p == 0)
    def _():
        a = adj_ref[0].astype(jnp.bfloat16)
        cache_scr[pl.ds(i * BM, BM), :] = a
        t = jnp.dot(a, s1_scr[...], preferred_element_type=jnp.float32)
        h1 = jnp.maximum(t + b1_ref[...], 0.0)
        s2_scr[pl.ds(i * BM, BM), :] = jnp.dot(
            h1, w2_ref[...], preferred_element_type=jnp.float32
        ).astype(jnp.bfloat16)
        gh_part = jnp.sum(h1, axis=0, keepdims=True)[None, None]

        @pl.when(i == 0)
        def _():
            gh_ref[...] = gh_part

        @pl.when(i != 0)
        def _():
            gh_ref[...] += gh_part

    @pl.when(p == 1)
    def _():
        a = cache_scr[pl.ds(i * BM, BM), :]
        t = jnp.dot(a, s2_scr[...], preferred_element_type=jnp.float32)
        h2 = jnp.maximum(t + b2_ref[...], 0.0)
        h2_ref[...] = h2[None]
        gh_part = jnp.sum(h2, axis=0, keepdims=True)[None, None]

        @pl.when(i == 0)
        def _():
            gh_ref[...] = gh_part

        @pl.when(i != 0)
        def _():
            gh_ref[...] += gh_part


@functools.partial(jax.jit, static_argnames=("interpret",))
def _encoder(x, adj, W1, b1, W2, b2, interpret=False):
    b1r = b1.reshape(1, H)
    b2r = b2.reshape(1, H)

    h2, gh = pl.pallas_call(
        _fused_kernel,
        grid=(B, 2, NUM_I),
        in_specs=[
            pl.BlockSpec(
                (1, BM, N),
                lambda b, p, i: (b, jnp.where(p == 0, i, NUM_I - 1), 0),
            ),
            pl.BlockSpec((1, N, F), lambda b, p, i: (b, 0, 0)),
            pl.BlockSpec((F, H), lambda b, p, i: (0, 0)),
            pl.BlockSpec((1, H), lambda b, p, i: (0, 0)),
            pl.BlockSpec((H, H), lambda b, p, i: (0, 0)),
            pl.BlockSpec((1, H), lambda b, p, i: (0, 0)),
        ],
        out_specs=[
            pl.BlockSpec(
                (1, BM, H),
                lambda b, p, i: (b, jnp.where(p == 0, 0, i), 0),
            ),
            pl.BlockSpec((1, 1, 1, H), lambda b, p, i: (b, p, 0, 0)),
        ],
        out_shape=[
            jax.ShapeDtypeStruct((B, N, H), jnp.float32),
            jax.ShapeDtypeStruct((B, 2, 1, H), jnp.float32),
        ],
        scratch_shapes=[
            pltpu.VMEM((N, H), jnp.bfloat16),
            pltpu.VMEM((N, H), jnp.bfloat16),
            pltpu.VMEM((N, N), jnp.bfloat16),
        ],
        compiler_params=pltpu.CompilerParams(
            dimension_semantics=("arbitrary", "arbitrary", "arbitrary"),
            vmem_limit_bytes=100 * 1024 * 1024,
        ),
        interpret=interpret,
    )(adj, x, W1, b1r, W2, b2r)

    return h2, gh.reshape(B, 2 * H)


def kernel(x, adj, W1, b1, W2, b2):
    return _encoder(x, adj, W1, b1, W2, b2)
